# baseline (speedup 1.0000x reference)
import functools

import jax
import jax.numpy as jnp
from jax import lax
from jax.experimental import pallas as pl
from jax.experimental.pallas import tpu as pltpu
from jax.experimental.pallas import tpu_sc as plsc

_LANES = 16
_NC = 2
_NS = 16
_NW = _NC * _NS
_RING = 12


def kernel(user_id, item_id, users_table, items_table):
    batch = user_id.shape[0]
    vocab, latent = users_table.shape
    bpw = batch // _NW

    uid = user_id.astype(jnp.int32)
    iid = item_id.astype(jnp.int32)
    ut_t = users_table.T
    it_t = items_table.T

    @functools.partial(
        pl.kernel,
        out_type=jax.ShapeDtypeStruct((batch,), jnp.float32),
        mesh=plsc.VectorSubcoreMesh(core_axis_name="c", subcore_axis_name="s"),
        compiler_params=pltpu.CompilerParams(
            needs_layout_passes=False, use_tc_tiling_on_sc=True),
        scratch_types=[
            pltpu.VMEM((bpw,), jnp.int32),
            pltpu.VMEM((bpw,), jnp.int32),
            pltpu.SMEM((bpw,), jnp.float32),
            pltpu.VMEM_SHARED((_NS, _RING, latent, 128), jnp.float32),
            pltpu.VMEM_SHARED((_NS, _RING, latent, 128), jnp.float32),
            pltpu.VMEM((latent, 128), jnp.float32),
            pltpu.VMEM((latent, 128), jnp.float32),
            pltpu.VMEM((bpw,), jnp.float32),
            pltpu.SemaphoreType.DMA((_RING,)),
            pltpu.SemaphoreType.DMA((_RING,)),
        ],
    )
    def mf(uid_hbm, iid_hbm, ut_hbm, it_hbm, out_hbm,
           uids, iids, outs, uwin, vwin, ucol, vcol, outv, usem, vsem):
        cid = lax.axis_index("c")
        sid = lax.axis_index("s")
        wid = sid * _NC + cid
        base = wid * bpw
        pltpu.sync_copy(uid_hbm.at[pl.ds(base, bpw)], uids)
        pltpu.sync_copy(iid_hbm.at[pl.ds(base, bpw)], iids)

        lane = lax.iota(jnp.int32, _LANES)
        lane_hi = lane + _LANES
        zero = jnp.zeros((_LANES,), jnp.int32)
        zcol = jnp.zeros((_LANES,), jnp.int32)

        def scalar_at(vec, mask):
            return jnp.sum(jnp.where(mask, vec, zero))

        def process_batch(i0, nb):
            copies = []
            lanes_u = []
            lanes_v = []
            for b in range(nb):
                e = i0 + b
                vbase = (e // _LANES) * _LANES
                uvec = uids[pl.ds(vbase, _LANES)]
                vvec = iids[pl.ds(vbase, _LANES)]
                mask = lane == (e % _LANES)
                u = scalar_at(uvec, mask)
                v = scalar_at(vvec, mask)
                uoff = pl.multiple_of(
                    lax.shift_left(lax.shift_right_logical(u, 7), 7), 128)
                voff = pl.multiple_of(
                    lax.shift_left(lax.shift_right_logical(v, 7), 7), 128)
                lanes_u.append(jnp.bitwise_and(u, 127))
                lanes_v.append(jnp.bitwise_and(v, 127))
                copies.append((
                    pltpu.async_copy(
                        ut_hbm.at[:, pl.ds(uoff, 128)], uwin.at[sid, b],
                        usem.at[b]),
                    pltpu.async_copy(
                        it_hbm.at[:, pl.ds(voff, 128)], vwin.at[sid, b],
                        vsem.at[b]),
                ))

            for b in range(nb):
                cu, cv = copies[b]
                cu.wait()
                cv.wait()
                pltpu.sync_copy(uwin.at[sid, b], ucol)
                pltpu.sync_copy(vwin.at[sid, b], vcol)
                ulv = jnp.full((_LANES,), lanes_u[b])
                vlv = jnp.full((_LANES,), lanes_v[b])
                ulo = plsc.load_gather(ucol, [lane, ulv])
                uhi = plsc.load_gather(ucol, [lane_hi, ulv])
                vlo = plsc.load_gather(vcol, [lane, vlv])
                vhi = plsc.load_gather(vcol, [lane_hi, vlv])
                prod = ulo * vlo + uhi * vhi
                outs[i0 + b] = jnp.sum(prod)

        def body(g, carry):
            process_batch(g * _RING, _RING)
            return carry

        n_full = bpw // _RING
        lax.fori_loop(0, n_full, body, 0)
        if bpw % _RING:
            process_batch(n_full * _RING, bpw % _RING)

        def pack(g, carry):
            vals = jnp.zeros((_LANES,), jnp.float32)
            for j in range(_LANES):
                s = outs[g * _LANES + j]
                vals = jnp.where(lane == j, jnp.full((_LANES,), s), vals)
            outv[pl.ds(g * _LANES, _LANES)] = vals
            return carry

        lax.fori_loop(0, bpw // _LANES, pack, 0)
        pltpu.sync_copy(outv, out_hbm.at[pl.ds(base, bpw)])

    return mf(uid, iid, ut_t, it_t)


# speedup vs baseline: 1.4674x; 1.4674x over previous
import functools

import jax
import jax.numpy as jnp
from jax import lax
from jax.experimental import pallas as pl
from jax.experimental.pallas import tpu as pltpu
from jax.experimental.pallas import tpu_sc as plsc

_LANES = 16
_NC = 2
_NS = 16
_NW = _NC * _NS
_RING = 15


def kernel(user_id, item_id, users_table, items_table):
    batch = user_id.shape[0]
    vocab, latent = users_table.shape
    bpw = batch // _NW

    uid = user_id.astype(jnp.int32)
    iid = item_id.astype(jnp.int32)
    ut_t = users_table.T
    it_t = items_table.T

    @functools.partial(
        pl.kernel,
        out_type=jax.ShapeDtypeStruct((batch,), jnp.float32),
        mesh=plsc.VectorSubcoreMesh(core_axis_name="c", subcore_axis_name="s"),
        compiler_params=pltpu.CompilerParams(
            needs_layout_passes=False, use_tc_tiling_on_sc=True),
        scratch_types=[
            pltpu.VMEM((bpw,), jnp.int32),
            pltpu.VMEM((bpw,), jnp.int32),
            pltpu.SMEM((bpw,), jnp.float32),
            pltpu.VMEM((_RING, latent, 128), jnp.float32),
            pltpu.VMEM((_RING, latent, 128), jnp.float32),
            pltpu.VMEM((bpw,), jnp.float32),
            pltpu.SemaphoreType.DMA((_RING,)),
            pltpu.SemaphoreType.DMA((_RING,)),
        ],
    )
    def mf(uid_hbm, iid_hbm, ut_hbm, it_hbm, out_hbm,
           uids, iids, outs, uwin, vwin, outv, usem, vsem):
        wid = lax.axis_index("s") * _NC + lax.axis_index("c")
        base = wid * bpw
        pltpu.sync_copy(uid_hbm.at[pl.ds(base, bpw)], uids)
        pltpu.sync_copy(iid_hbm.at[pl.ds(base, bpw)], iids)

        lane = lax.iota(jnp.int32, _LANES)
        lane_hi = lane + _LANES
        zero = jnp.zeros((_LANES,), jnp.int32)

        def scalar_at(vec, mask):
            return jnp.sum(jnp.where(mask, vec, zero))

        def process_batch(i0, nb):
            copies = []
            lanes_u = []
            lanes_v = []
            for b in range(nb):
                e = i0 + b
                vbase = (e // _LANES) * _LANES
                uvec = uids[pl.ds(vbase, _LANES)]
                vvec = iids[pl.ds(vbase, _LANES)]
                mask = lane == (e % _LANES)
                u = scalar_at(uvec, mask)
                v = scalar_at(vvec, mask)
                uoff = pl.multiple_of(
                    lax.shift_left(lax.shift_right_logical(u, 7), 7), 128)
                voff = pl.multiple_of(
                    lax.shift_left(lax.shift_right_logical(v, 7), 7), 128)
                lanes_u.append(jnp.full((_LANES,), jnp.bitwise_and(u, 127)))
                lanes_v.append(jnp.full((_LANES,), jnp.bitwise_and(v, 127)))
                copies.append((
                    pltpu.async_copy(
                        ut_hbm.at[:, pl.ds(uoff, 128)], uwin.at[b],
                        usem.at[b]),
                    pltpu.async_copy(
                        it_hbm.at[:, pl.ds(voff, 128)], vwin.at[b],
                        vsem.at[b]),
                ))

            for b in range(nb):
                cu, cv = copies[b]
                cu.wait()
                cv.wait()
                bb = jnp.full((_LANES,), b, jnp.int32)
                ulo = plsc.load_gather(uwin, [bb, lane, lanes_u[b]])
                uhi = plsc.load_gather(uwin, [bb, lane_hi, lanes_u[b]])
                vlo = plsc.load_gather(vwin, [bb, lane, lanes_v[b]])
                vhi = plsc.load_gather(vwin, [bb, lane_hi, lanes_v[b]])
                prod = ulo * vlo + uhi * vhi
                outs[i0 + b] = jnp.sum(prod)

        def body(g, carry):
            process_batch(g * _RING, _RING)
            return carry

        n_full = bpw // _RING
        lax.fori_loop(0, n_full, body, 0)
        if bpw % _RING:
            process_batch(n_full * _RING, bpw % _RING)

        def pack(g, carry):
            vals = jnp.zeros((_LANES,), jnp.float32)
            for j in range(_LANES):
                s = outs[g * _LANES + j]
                vals = jnp.where(lane == j, jnp.full((_LANES,), s), vals)
            outv[pl.ds(g * _LANES, _LANES)] = vals
            return carry

        lax.fori_loop(0, bpw // _LANES, pack, 0)
        pltpu.sync_copy(outv, out_hbm.at[pl.ds(base, bpw)])

    return mf(uid, iid, ut_t, it_t)
